# Initial kernel scaffold; baseline (speedup 1.0000x reference)
import jax
import jax.numpy as jnp
from jax.experimental import pallas as pl


def kernel(x, tables):
    raise NotImplementedError("write your pallas kernel here")


# speedup vs baseline: 379.4548x; 379.4548x over previous
import jax
import jax.numpy as jnp
from jax.experimental import pallas as pl
from jax.experimental.pallas import tpu as pltpu

_BLOCK_ROWS = 2000


def _body(x_ref, t_ref, o_ref):
    t0 = t_ref[0]
    t1 = t_ref[1]
    diff = t1 - t0
    base = jnp.sum(t0, axis=0, keepdims=True)
    xb = x_ref[...].astype(jnp.float32)
    acc = jax.lax.dot_general(
        xb, diff,
        dimension_numbers=(((1,), (0,)), ((), ())),
        preferred_element_type=jnp.float32,
    )
    o_ref[...] = acc + base


def kernel(x, tables):
    n, f = x.shape
    d = tables.shape[-1]
    tt = tables.transpose(1, 0, 2)
    grid = (n + _BLOCK_ROWS - 1) // _BLOCK_ROWS
    return pl.pallas_call(
        _body,
        grid=(grid,),
        in_specs=[
            pl.BlockSpec((_BLOCK_ROWS, f), lambda i: (i, 0)),
            pl.BlockSpec((2, f, d), lambda i: (0, 0, 0)),
        ],
        out_specs=pl.BlockSpec((_BLOCK_ROWS, d), lambda i: (i, 0)),
        out_shape=jax.ShapeDtypeStruct((n, d), jnp.float32),
    )(x, tt)


# speedup vs baseline: 497.6454x; 1.3115x over previous
import jax
import jax.numpy as jnp
from jax.experimental import pallas as pl
from jax.experimental.pallas import tpu as pltpu

_BLOCK_ROWS = 10000


def _body(x_ref, t_ref, o_ref):
    t0 = t_ref[0]
    t1 = t_ref[1]
    diff = t1 - t0
    base = jnp.sum(t0, axis=0, keepdims=True)
    xb = x_ref[...].astype(jnp.float32)
    acc = jax.lax.dot_general(
        xb, diff,
        dimension_numbers=(((1,), (0,)), ((), ())),
        preferred_element_type=jnp.float32,
    )
    o_ref[...] = acc + base


def kernel(x, tables):
    n, f = x.shape
    d = tables.shape[-1]
    tt = tables.transpose(1, 0, 2)
    grid = (n + _BLOCK_ROWS - 1) // _BLOCK_ROWS
    return pl.pallas_call(
        _body,
        grid=(grid,),
        in_specs=[
            pl.BlockSpec((_BLOCK_ROWS, f), lambda i: (i, 0)),
            pl.BlockSpec((2, f, d), lambda i: (0, 0, 0)),
        ],
        out_specs=pl.BlockSpec((_BLOCK_ROWS, d), lambda i: (i, 0)),
        out_shape=jax.ShapeDtypeStruct((n, d), jnp.float32),
    )(x, tt)
